# Initial kernel scaffold; baseline (speedup 1.0000x reference)
import jax
import jax.numpy as jnp
from jax.experimental import pallas as pl


def kernel(coord, coord_gt, coord_mask, lc_ind, lc_span, ct_mask):
    raise NotImplementedError("write your pallas kernel here")


# speedup vs baseline: 4.5398x; 4.5398x over previous
import functools

import jax
import jax.numpy as jnp
from jax import lax
from jax.experimental import pallas as pl
from jax.experimental.pallas import tpu as pltpu
from jax.experimental.pallas import tpu_sc as plsc

EPS = 0.0001
B, C, H, W, N = 32, 2, 512, 512, 1000
HW = H * W
NPAD = 1024
CHUNK = 128
NCHUNK = NPAD // CHUNK
NJ = 8
LANES = 16

NC, NS = 2, 16
NW = NC * NS


ROWS = B * C * H
BLK = 2048


def _dense_body(c_ref, g_ref, m_ref, sabs_ref, smask_ref):
    i = pl.program_id(0)

    @pl.when(i == 0)
    def _():
        sabs_ref[0, 0] = 0.0
        smask_ref[0, 0] = 0.0

    m = m_ref[...]
    sabs_ref[0, 0] += jnp.sum(jnp.abs(c_ref[...] * m - g_ref[...] * m))
    smask_ref[0, 0] += jnp.sum(m)


def _dense_sums(coord2d, gt2d, mask2d):
    sabs, smask = pl.pallas_call(
        _dense_body,
        grid=(ROWS // BLK,),
        in_specs=[pl.BlockSpec((BLK, W), lambda i: (i, 0))] * 3,
        out_specs=[
            pl.BlockSpec(memory_space=pltpu.SMEM),
            pl.BlockSpec(memory_space=pltpu.SMEM),
        ],
        out_shape=[jax.ShapeDtypeStruct((1, 1), jnp.float32)] * 2,
    )(coord2d, gt2d, mask2d)
    return sabs[0, 0], smask[0, 0]


@functools.cache
def _sc_span_kernel_fn():
    mesh = plsc.VectorSubcoreMesh(
        core_axis_name="c", subcore_axis_name="s", num_cores=NC,
        num_subcores=NS,
    )
    return pl.kernel(
        _sc_span_body,
        out_type=jax.ShapeDtypeStruct((B, 2, LANES), jnp.float32),
        mesh=mesh,
        scratch_types=[
            pltpu.VMEM((NJ, NCHUNK, CHUNK), jnp.int32),
            pltpu.VMEM((NJ, NPAD), jnp.float32),
            pltpu.VMEM((NPAD,), jnp.float32),
            pltpu.VMEM((NPAD,), jnp.float32),
            pltpu.VMEM((NPAD,), jnp.float32),
            pltpu.VMEM((2, LANES), jnp.float32),
            pltpu.SemaphoreType.DMA,
        ],
    )


def _sc_span_body(idx_hbm, coord_hbm, m_hbm, gc_hbm, gr_hbm, out_hbm,
                  idx_v, gath_v, m_v, gc_v, gr_v, res_v, sem):
    b = lax.axis_index("s") * NC + lax.axis_index("c")

    pltpu.sync_copy(idx_hbm.at[b], idx_v)
    pltpu.sync_copy(m_hbm.at[b], m_v)
    pltpu.sync_copy(gc_hbm.at[b], gc_v)
    pltpu.sync_copy(gr_hbm.at[b], gr_v)

    copies = []
    for j in range(NJ):
        for c in range(NCHUNK):
            copies.append(pltpu.async_copy(
                coord_hbm.at[idx_v.at[j, c]],
                gath_v.at[j, pl.ds(c * CHUNK, CHUNK)],
                sem,
            ))
    for cp in copies:
        cp.wait()

    def step(k, carry):
        acc, macc = carry
        sl = pl.ds(k * LANES, LANES)
        m = m_v[sl]
        gc = gc_v[sl] * m
        gr = gr_v[sl] * m
        c0 = gath_v[0, sl]
        c1 = gath_v[1, sl]
        c2 = gath_v[2, sl]
        c3 = gath_v[3, sl]
        r0 = gath_v[4, sl]
        r1 = gath_v[5, sl]
        r2 = gath_v[6, sl]
        r3 = gath_v[7, sl]
        acc = acc + (jnp.abs((c1 - c0) * m - gc) + jnp.abs((c2 - c3) * m - gc)
                     + jnp.abs((r3 - r0) * m - gr) + jnp.abs((r2 - r1) * m - gr))
        macc = macc + m
        return acc, macc

    zero = jnp.zeros((LANES,), jnp.float32)
    acc, macc = lax.fori_loop(0, NPAD // LANES, step, (zero, zero))
    res_v[0, :] = acc
    res_v[1, :] = macc
    pltpu.sync_copy(res_v, out_hbm.at[b])


def kernel(coord, coord_gt, coord_mask, lc_ind, lc_span, ct_mask):
    coord2d = coord.reshape(ROWS, W)
    gt2d = coord_gt.reshape(ROWS, W)
    mask2d = coord_mask.reshape(ROWS, W)
    sabs, smask = _dense_sums(coord2d, gt2d, mask2d)
    coord_loss = sabs / (smask + EPS)

    ind = lc_ind.astype(jnp.int32)
    base = (jnp.arange(B, dtype=jnp.int32) * (C * HW))[:, None, None]
    col_idx = base + ind
    row_idx = col_idx + HW
    pad = ((0, 0), (0, NPAD - N), (0, 0))
    colp = jnp.pad(col_idx, pad).transpose(0, 2, 1)
    rowp = jnp.pad(row_idx, pad).transpose(0, 2, 1)
    idx_all = jnp.concatenate([colp, rowp], axis=1)
    idx_all = idx_all.reshape(B, NJ, NCHUNK, CHUNK)

    m_pad = jnp.pad(ct_mask, ((0, 0), (0, NPAD - N)))
    gc_pad = jnp.pad(lc_span[..., 0], ((0, 0), (0, NPAD - N)))
    gr_pad = jnp.pad(lc_span[..., 1], ((0, 0), (0, NPAD - N)))

    partials = _sc_span_kernel_fn()(idx_all, coord.reshape(-1), m_pad, gc_pad,
                                    gr_pad)
    span_sum = jnp.sum(partials[:, 0, :])
    msum = jnp.sum(partials[:, 1, :])
    span_diff_loss = span_sum / (2.0 * msum + EPS)
    return (coord_loss, span_diff_loss)
